# Initial kernel scaffold; baseline (speedup 1.0000x reference)
import jax
import jax.numpy as jnp
from jax import lax
from jax.experimental import pallas as pl
from jax.experimental.pallas import tpu as pltpu
from jax.experimental.pallas import tpu_sc as plsc

N = 10000
E = 320000
F = 128
NC = 2
NS = 16
NW = NC * NS
N_PAD = 10240
ROWS_PER_SUB = N_PAD // NS
EPW = E // NW
CHUNK = 128
NFULL = EPW // CHUNK
TAIL = EPW - NFULL * CHUNK
DEG_W = 16


def _agg_body(with_deg):
    def body(*refs):
        if with_deg:
            (feat_hbm, src_hbm, dst_hbm, zeros_hbm, zeros16_hbm, ones_hbm,
             out_hbm, degout_hbm,
             src_v, dst_v, src_t, dst_t, rows_v, rows_t, ones_v,
             acc, dacc, sem) = refs
        else:
            (feat_hbm, src_hbm, dst_hbm, zeros_hbm,
             out_hbm,
             src_v, dst_v, src_t, dst_t, rows_v, rows_t,
             acc, sem) = refs
        c = lax.axis_index("c")
        s = lax.axis_index("s")
        wid = s * NC + c
        r0 = s * ROWS_PER_SUB
        pltpu.sync_copy(zeros_hbm.at[pl.ds(r0, ROWS_PER_SUB)],
                        acc.at[pl.ds(r0, ROWS_PER_SUB)])
        if with_deg:
            pltpu.sync_copy(zeros16_hbm.at[pl.ds(r0, ROWS_PER_SUB)],
                            dacc.at[pl.ds(r0, ROWS_PER_SUB)])
            pltpu.sync_copy(ones_hbm, ones_v)
        plsc.subcore_barrier()

        base = wid * EPW

        def step(i, carry):
            off = base + i * CHUNK
            pltpu.sync_copy(src_hbm.at[pl.ds(off, CHUNK)], src_v)
            pltpu.sync_copy(dst_hbm.at[pl.ds(off, CHUNK)], dst_v)
            pltpu.async_copy(feat_hbm.at[src_v], rows_v, sem).wait()
            pltpu.sync_copy(rows_v, acc.at[dst_v], add=True)
            if with_deg:
                pltpu.sync_copy(ones_v, dacc.at[dst_v], add=True)
            return carry

        lax.fori_loop(0, NFULL, step, 0)
        if TAIL:
            off = base + NFULL * CHUNK
            pltpu.sync_copy(src_hbm.at[pl.ds(off, TAIL)], src_t)
            pltpu.sync_copy(dst_hbm.at[pl.ds(off, TAIL)], dst_t)
            pltpu.async_copy(feat_hbm.at[src_t], rows_t, sem).wait()
            pltpu.sync_copy(rows_t, acc.at[dst_t], add=True)
            if with_deg:
                pltpu.sync_copy(ones_v.at[pl.ds(0, TAIL)], dacc.at[dst_t],
                                add=True)
        plsc.subcore_barrier()

        pltpu.sync_copy(acc.at[pl.ds(r0, ROWS_PER_SUB)],
                        out_hbm.at[c, pl.ds(r0, ROWS_PER_SUB)])
        if with_deg:
            pltpu.sync_copy(dacc.at[pl.ds(r0, ROWS_PER_SUB)],
                            degout_hbm.at[c, pl.ds(r0, ROWS_PER_SUB)])
    return body


def _make_agg(with_deg):
    scratch = [
        pltpu.VMEM((CHUNK,), jnp.int32),
        pltpu.VMEM((CHUNK,), jnp.int32),
        pltpu.VMEM((TAIL,), jnp.int32),
        pltpu.VMEM((TAIL,), jnp.int32),
        pltpu.VMEM((CHUNK, F), jnp.float32),
        pltpu.VMEM((TAIL, F), jnp.float32),
    ]
    if with_deg:
        scratch += [
            pltpu.VMEM((CHUNK, DEG_W), jnp.float32),
            pltpu.VMEM_SHARED((N_PAD, F), jnp.float32),
            pltpu.VMEM_SHARED((N_PAD, DEG_W), jnp.float32),
        ]
        out_type = (jax.ShapeDtypeStruct((NC, N_PAD, F), jnp.float32),
                    jax.ShapeDtypeStruct((NC, N_PAD, DEG_W), jnp.float32))
    else:
        scratch += [pltpu.VMEM_SHARED((N_PAD, F), jnp.float32)]
        out_type = jax.ShapeDtypeStruct((NC, N_PAD, F), jnp.float32)
    scratch += [pltpu.SemaphoreType.DMA]
    return pl.kernel(
        _agg_body(with_deg),
        out_type=out_type,
        mesh=plsc.VectorSubcoreMesh(core_axis_name="c", subcore_axis_name="s"),
        scratch_types=scratch,
        name="sage_agg_deg" if with_deg else "sage_agg",
    )


_agg_deg_call = _make_agg(True)
_agg_call = _make_agg(False)

BLK = 1000


def _combine_body(relu):
    def body(x_ref, p0_ref, p1_ref, d0_ref, d1_ref, ws_ref, wn_ref, b_ref,
             out_ref):
        agg = p0_ref[...] + p1_ref[...]
        deg = d0_ref[...][:, 0:1] + d1_ref[...][:, 0:1]
        mean = agg * (1.0 / jnp.maximum(deg, 1.0))
        y = (jnp.dot(x_ref[...], ws_ref[...],
                     preferred_element_type=jnp.float32)
             + jnp.dot(mean, wn_ref[...], preferred_element_type=jnp.float32)
             + b_ref[...])
        out_ref[...] = jnp.maximum(y, 0.0) if relu else y
    return body


def _combine(x, p0, p1, d0, d1, Ws, Wn, b, relu):
    return pl.pallas_call(
        _combine_body(relu),
        out_shape=jax.ShapeDtypeStruct((N, F), jnp.float32),
        grid=(N // BLK,),
        in_specs=[
            pl.BlockSpec((BLK, F), lambda i: (i, 0)),
            pl.BlockSpec((BLK, F), lambda i: (i, 0)),
            pl.BlockSpec((BLK, F), lambda i: (i, 0)),
            pl.BlockSpec((BLK, DEG_W), lambda i: (i, 0)),
            pl.BlockSpec((BLK, DEG_W), lambda i: (i, 0)),
            pl.BlockSpec((F, F), lambda i: (0, 0)),
            pl.BlockSpec((F, F), lambda i: (0, 0)),
            pl.BlockSpec((1, F), lambda i: (0, 0)),
        ],
        out_specs=pl.BlockSpec((BLK, F), lambda i: (i, 0)),
    )(x, p0, p1, d0, d1, Ws, Wn, b.reshape(1, F))


def kernel(x, edge_index, W_self1, W_neigh1, b1, W_self2, W_neigh2, b2):
    src = edge_index[0]
    dst = edge_index[1]
    zeros_hbm = jnp.zeros((N_PAD, F), jnp.float32)
    zeros16_hbm = jnp.zeros((N_PAD, DEG_W), jnp.float32)
    ones_hbm = jnp.ones((CHUNK, DEG_W), jnp.float32)

    parts1, degp = _agg_deg_call(x, src, dst, zeros_hbm, zeros16_hbm,
                                 ones_hbm)
    d0 = degp[0, :N]
    d1 = degp[1, :N]
    h = _combine(x, parts1[0, :N], parts1[1, :N], d0, d1,
                 W_self1, W_neigh1, b1, relu=True)
    parts2 = _agg_call(h, src, dst, zeros_hbm)
    out = _combine(h, parts2[0, :N], parts2[1, :N], d0, d1,
                   W_self2, W_neigh2, b2, relu=False)
    return out


# speedup vs baseline: 6.9168x; 6.9168x over previous
import jax
import jax.numpy as jnp
from jax import lax
from jax.experimental import pallas as pl
from jax.experimental.pallas import tpu as pltpu
from jax.experimental.pallas import tpu_sc as plsc

N = 10000
E = 320000
F = 128
NC = 2
NS = 16
NW = NC * NS
N_PAD = 10240
ROWS_PER_SUB = N_PAD // NS
EPW = E // NW
CHUNK = 128
NFULL = EPW // CHUNK
TAIL = EPW - NFULL * CHUNK
L = 16


def _agg_body(with_deg):
    def body(*refs):
        if with_deg:
            (feat_hbm, src_hbm, dst_hbm, zeros_hbm,
             out0_hbm, out1_hbm, deg_hbm,
             src_v, dst_v, src_t, dst_t, rows_v, rows_t,
             hist, acc, sem) = refs
        else:
            (feat_hbm, src_hbm, dst_hbm, zeros_hbm,
             out0_hbm, out1_hbm,
             src_v, dst_v, src_t, dst_t, rows_v, rows_t,
             acc, sem) = refs
        c = lax.axis_index("c")
        s = lax.axis_index("s")
        wid = s * NC + c
        r0 = s * ROWS_PER_SUB
        pltpu.sync_copy(zeros_hbm.at[pl.ds(r0, ROWS_PER_SUB)],
                        acc.at[pl.ds(r0, ROWS_PER_SUB)])
        if with_deg:
            def zstep(i, carry):
                hist[pl.ds(i * L, L)] = jnp.zeros((L,), jnp.float32)
                return carry
            lax.fori_loop(0, N_PAD // L, zstep, 0)
        plsc.subcore_barrier()

        base = wid * EPW

        def step(i, carry):
            off = base + i * CHUNK
            pltpu.sync_copy(src_hbm.at[pl.ds(off, CHUNK)], src_v)
            pltpu.sync_copy(dst_hbm.at[pl.ds(off, CHUNK)], dst_v)
            pltpu.async_copy(feat_hbm.at[src_v], rows_v, sem).wait()
            pltpu.sync_copy(rows_v, acc.at[dst_v], add=True)
            if with_deg:
                for j in range(CHUNK // L):
                    idx = dst_v[pl.ds(j * L, L)]
                    plsc.addupdate_scatter(hist, [idx],
                                           jnp.ones((L,), jnp.float32))
            return carry

        lax.fori_loop(0, NFULL, step, 0)
        if TAIL:
            off = base + NFULL * CHUNK
            pltpu.sync_copy(src_hbm.at[pl.ds(off, TAIL)], src_t)
            pltpu.sync_copy(dst_hbm.at[pl.ds(off, TAIL)], dst_t)
            pltpu.async_copy(feat_hbm.at[src_t], rows_t, sem).wait()
            pltpu.sync_copy(rows_t, acc.at[dst_t], add=True)
            if with_deg:
                for j in range(TAIL // L):
                    idx = dst_t[pl.ds(j * L, L)]
                    plsc.addupdate_scatter(hist, [idx],
                                           jnp.ones((L,), jnp.float32))
        plsc.subcore_barrier()

        @pl.when(c == 0)
        def _():
            pltpu.sync_copy(acc.at[pl.ds(r0, ROWS_PER_SUB)],
                            out0_hbm.at[pl.ds(r0, ROWS_PER_SUB)])

        @pl.when(c == 1)
        def _():
            pltpu.sync_copy(acc.at[pl.ds(r0, ROWS_PER_SUB)],
                            out1_hbm.at[pl.ds(r0, ROWS_PER_SUB)])

        if with_deg:
            pltpu.sync_copy(hist, deg_hbm.at[wid])
    return body


def _make_agg(with_deg):
    scratch = [
        pltpu.VMEM((CHUNK,), jnp.int32),
        pltpu.VMEM((CHUNK,), jnp.int32),
        pltpu.VMEM((TAIL,), jnp.int32),
        pltpu.VMEM((TAIL,), jnp.int32),
        pltpu.VMEM((CHUNK, F), jnp.float32),
        pltpu.VMEM((TAIL, F), jnp.float32),
    ]
    out_type = [jax.ShapeDtypeStruct((N_PAD, F), jnp.float32),
                jax.ShapeDtypeStruct((N_PAD, F), jnp.float32)]
    if with_deg:
        scratch += [pltpu.VMEM((N_PAD,), jnp.float32)]
        out_type += [jax.ShapeDtypeStruct((NW, N_PAD), jnp.float32)]
    scratch += [pltpu.VMEM_SHARED((N_PAD, F), jnp.float32),
                pltpu.SemaphoreType.DMA]
    return pl.kernel(
        _agg_body(with_deg),
        out_type=tuple(out_type),
        mesh=plsc.VectorSubcoreMesh(core_axis_name="c", subcore_axis_name="s"),
        scratch_types=scratch,
        compiler_params=pltpu.CompilerParams(needs_layout_passes=False),
        name="sage_agg_deg" if with_deg else "sage_agg",
    )


_agg_deg_call = _make_agg(True)
_agg_call = _make_agg(False)

BLK = 1280


def _combine_body(relu):
    def body(x_ref, p0_ref, p1_ref, dp_ref, ws_ref, wn_ref, b_ref, out_ref):
        agg = p0_ref[...] + p1_ref[...]
        deg = jnp.sum(dp_ref[...], axis=0).reshape(BLK, 1)
        mean = agg * (1.0 / jnp.maximum(deg, 1.0))
        y = (jnp.dot(x_ref[...], ws_ref[...],
                     preferred_element_type=jnp.float32)
             + jnp.dot(mean, wn_ref[...], preferred_element_type=jnp.float32)
             + b_ref[...])
        out_ref[...] = jnp.maximum(y, 0.0) if relu else y
    return body


def _combine(x, p0, p1, degparts, Ws, Wn, b, relu):
    return pl.pallas_call(
        _combine_body(relu),
        out_shape=jax.ShapeDtypeStruct((N_PAD, F), jnp.float32),
        grid=(N_PAD // BLK,),
        in_specs=[
            pl.BlockSpec((BLK, F), lambda i: (i, 0)),
            pl.BlockSpec((BLK, F), lambda i: (i, 0)),
            pl.BlockSpec((BLK, F), lambda i: (i, 0)),
            pl.BlockSpec((NW, BLK), lambda i: (0, i)),
            pl.BlockSpec((F, F), lambda i: (0, 0)),
            pl.BlockSpec((F, F), lambda i: (0, 0)),
            pl.BlockSpec((1, F), lambda i: (0, 0)),
        ],
        out_specs=pl.BlockSpec((BLK, F), lambda i: (i, 0)),
    )(x, p0, p1, degparts, Ws, Wn, b.reshape(1, F))


def kernel(x, edge_index, W_self1, W_neigh1, b1, W_self2, W_neigh2, b2):
    src = edge_index[0]
    dst = edge_index[1]
    x_pad = jnp.pad(x, ((0, N_PAD - N), (0, 0)))
    zeros_hbm = jnp.zeros((N_PAD, F), jnp.float32)

    p0, p1, degparts = _agg_deg_call(x_pad, src, dst, zeros_hbm)
    h = _combine(x_pad, p0, p1, degparts, W_self1, W_neigh1, b1, relu=True)
    q0, q1 = _agg_call(h, src, dst, zeros_hbm)
    out = _combine(h, q0, q1, degparts, W_self2, W_neigh2, b2, relu=False)
    return out[:N]
